# Initial kernel scaffold; baseline (speedup 1.0000x reference)
import jax
import jax.numpy as jnp
from jax.experimental import pallas as pl


def kernel(x, W, n_active, capacity):
    raise NotImplementedError("write your pallas kernel here")


# speedup vs baseline: 4.7587x; 4.7587x over previous
import functools

import jax
import jax.numpy as jnp
from jax.experimental import pallas as pl
from jax.experimental.pallas import tpu as pltpu

D_MODEL = 768
N_EXPERTS = 64
BT = 1024


def _router_body(x_ref, wt_ref, gates_ref, idx_ref, loss_ref, counts_ref):
    i = pl.program_id(0)
    nblk = pl.num_programs(0)

    x = x_ref[...]
    wt = wt_ref[...]
    logits = jnp.dot(x, wt, preferred_element_type=jnp.float32)

    m = jnp.max(logits, axis=-1, keepdims=True)
    unnorm = jnp.exp(logits - m)
    probs = unnorm / jnp.sum(unnorm, axis=-1, keepdims=True)

    lane = jax.lax.broadcasted_iota(jnp.int32, probs.shape, 1)
    p1 = jnp.max(probs, axis=-1, keepdims=True)
    e1 = jnp.min(jnp.where(probs == p1, lane, N_EXPERTS), axis=-1, keepdims=True)
    probs2 = jnp.where(lane == e1, -jnp.inf, probs)
    p2 = jnp.max(probs2, axis=-1, keepdims=True)
    e2 = jnp.min(jnp.where(probs2 == p2, lane, N_EXPERTS), axis=-1, keepdims=True)

    denom = p1 + p2
    s1 = p1 / denom
    s2 = p2 / denom

    gates = jnp.where(lane == e1, s1, 0.0) + jnp.where(lane == e2, s2, 0.0)
    gates_ref[...] = gates
    idx_ref[...] = jnp.concatenate([e1, e2], axis=1)

    @pl.when(i == 0)
    def _init():
        counts_ref[...] = jnp.zeros_like(counts_ref)

    counts_ref[...] += jnp.sum(gates, axis=0, keepdims=True)

    @pl.when(i == nblk - 1)
    def _finish():
        counts = counts_ref[...]
        total = jnp.sum(counts)
        dev = counts / total * N_EXPERTS - 1.0
        loss_ref[...] = jnp.mean(dev * dev, axis=1, keepdims=True)


@functools.partial(jax.jit, static_argnums=())
def kernel(x, W, n_active, capacity):
    b, s, d = x.shape
    t = b * s
    xf = x.reshape(t, d)
    wt = W.T
    grid = (t // BT,)
    gates2d, idx2d, loss2d = pl.pallas_call(
        _router_body,
        grid=grid,
        in_specs=[
            pl.BlockSpec((BT, D_MODEL), lambda i: (i, 0)),
            pl.BlockSpec((D_MODEL, N_EXPERTS), lambda i: (0, 0)),
        ],
        out_specs=[
            pl.BlockSpec((BT, N_EXPERTS), lambda i: (i, 0)),
            pl.BlockSpec((BT, 2), lambda i: (i, 0)),
            pl.BlockSpec((1, 1), lambda i: (0, 0)),
        ],
        out_shape=[
            jax.ShapeDtypeStruct((t, N_EXPERTS), jnp.float32),
            jax.ShapeDtypeStruct((t, 2), jnp.int32),
            jax.ShapeDtypeStruct((1, 1), jnp.float32),
        ],
        scratch_shapes=[pltpu.VMEM((1, N_EXPERTS), jnp.float32)],
    )(xf, wt)
    gates = gates2d.reshape(b, s, N_EXPERTS)
    idx = idx2d.reshape(b, s, 2)
    return gates, idx, loss2d[0, 0]


# speedup vs baseline: 5.5785x; 1.1723x over previous
import functools

import jax
import jax.numpy as jnp
from jax.experimental import pallas as pl
from jax.experimental.pallas import tpu as pltpu

D_MODEL = 768
N_EXPERTS = 64
BT = 1024


def _router_body(x_ref, w_ref, gates_ref, idx_ref, loss_ref, counts_ref):
    i = pl.program_id(0)
    nblk = pl.num_programs(0)

    x = x_ref[...]
    w = w_ref[...]
    lt = jax.lax.dot_general(w, x, (((1,), (1,)), ((), ())),
                             preferred_element_type=jnp.float32)

    row = jax.lax.broadcasted_iota(jnp.int32, lt.shape, 0).astype(jnp.float32)
    m1 = jnp.max(lt, axis=0, keepdims=True)
    e1 = jnp.min(jnp.where(lt == m1, row, float(N_EXPERTS)),
                 axis=0, keepdims=True)
    lt2 = jnp.where(row == e1, -jnp.inf, lt)
    m2 = jnp.max(lt2, axis=0, keepdims=True)
    e2 = jnp.min(jnp.where(lt2 == m2, row, float(N_EXPERTS)),
                 axis=0, keepdims=True)

    ed = jnp.exp(m2 - m1)
    s1 = 1.0 / (1.0 + ed)
    s2 = ed * s1

    gates_t = (jnp.where(row == e1, s1, 0.0)
               + jnp.where(row == e2, s2, 0.0))
    gates_ref[...] = gates_t.T

    idx_t = jnp.concatenate([e1, e2], axis=0).astype(jnp.int32)
    idx_ref[...] = idx_t.T

    @pl.when(i == 0)
    def _init():
        counts_ref[...] = jnp.zeros_like(counts_ref)

    counts_ref[...] += jnp.sum(gates_t, axis=1, keepdims=True)

    @pl.when(i == nblk - 1)
    def _finish():
        counts = counts_ref[...]
        total = jnp.sum(counts)
        dev = counts / total * N_EXPERTS - 1.0
        loss_ref[...] = jnp.mean(dev * dev, axis=0, keepdims=True)


@functools.partial(jax.jit, static_argnums=())
def kernel(x, W, n_active, capacity):
    b, s, d = x.shape
    t = b * s
    xf = x.reshape(t, d)
    grid = (t // BT,)
    gates2d, idx2d, loss2d = pl.pallas_call(
        _router_body,
        grid=grid,
        in_specs=[
            pl.BlockSpec((BT, D_MODEL), lambda i: (i, 0)),
            pl.BlockSpec((N_EXPERTS, D_MODEL), lambda i: (0, 0)),
        ],
        out_specs=[
            pl.BlockSpec((BT, N_EXPERTS), lambda i: (i, 0)),
            pl.BlockSpec((BT, 2), lambda i: (i, 0)),
            pl.BlockSpec((1, 1), lambda i: (0, 0)),
        ],
        out_shape=[
            jax.ShapeDtypeStruct((t, N_EXPERTS), jnp.float32),
            jax.ShapeDtypeStruct((t, 2), jnp.int32),
            jax.ShapeDtypeStruct((1, 1), jnp.float32),
        ],
        scratch_shapes=[pltpu.VMEM((N_EXPERTS, 1), jnp.float32)],
    )(xf, W)
    gates = gates2d.reshape(b, s, N_EXPERTS)
    idx = idx2d.reshape(b, s, 2)
    return gates, idx, loss2d[0, 0]


# speedup vs baseline: 9.6600x; 1.7316x over previous
import functools

import jax
import jax.numpy as jnp
from jax.experimental import pallas as pl
from jax.experimental.pallas import tpu as pltpu

D_MODEL = 768
N_EXPERTS = 64
BT = 1024


def _router_body(x_ref, w_ref, gates_ref, idx_ref, loss_ref, counts_ref):
    i = pl.program_id(0)
    nblk = pl.num_programs(0)

    x = x_ref[...]
    w = w_ref[...]
    lt = jax.lax.dot_general(w, x, (((1,), (1,)), ((), ())),
                             preferred_element_type=jnp.float32)

    row = jax.lax.broadcasted_iota(jnp.int32, lt.shape, 0).astype(jnp.float32)
    m1 = jnp.max(lt, axis=0, keepdims=True)
    e1 = jnp.min(jnp.where(lt == m1, row, float(N_EXPERTS)),
                 axis=0, keepdims=True)
    lt2 = jnp.where(row == e1, -jnp.inf, lt)
    m2 = jnp.max(lt2, axis=0, keepdims=True)
    e2 = jnp.min(jnp.where(lt2 == m2, row, float(N_EXPERTS)),
                 axis=0, keepdims=True)

    ed = jnp.exp(m2 - m1)
    s1 = 1.0 / (1.0 + ed)
    s2 = ed * s1

    gates_t = (jnp.where(row == e1, s1, 0.0)
               + jnp.where(row == e2, s2, 0.0))
    gates_ref[...] = gates_t[None]

    idx_t = jnp.concatenate([e1, e2], axis=0).astype(jnp.int32)
    idx_ref[...] = idx_t[None]

    @pl.when(i == 0)
    def _init():
        counts_ref[...] = jnp.zeros_like(counts_ref)

    counts_ref[...] += jnp.sum(gates_t, axis=1, keepdims=True)

    @pl.when(i == nblk - 1)
    def _finish():
        counts = counts_ref[...]
        total = jnp.sum(counts)
        dev = counts / total * N_EXPERTS - 1.0
        loss_ref[...] = jnp.mean(dev * dev, axis=0, keepdims=True)


@functools.partial(jax.jit, static_argnums=())
def kernel(x, W, n_active, capacity):
    b, s, d = x.shape
    t = b * s
    blk_per_batch = s // BT
    xf = x.reshape(t, d)
    grid = (t // BT,)
    gates3, idx3, loss2d = pl.pallas_call(
        _router_body,
        grid=grid,
        in_specs=[
            pl.BlockSpec((BT, D_MODEL), lambda i: (i, 0)),
            pl.BlockSpec((N_EXPERTS, D_MODEL), lambda i: (0, 0)),
        ],
        out_specs=[
            pl.BlockSpec((1, N_EXPERTS, BT),
                         lambda i: (i // blk_per_batch, 0, i % blk_per_batch)),
            pl.BlockSpec((1, 2, BT),
                         lambda i: (i // blk_per_batch, 0, i % blk_per_batch)),
            pl.BlockSpec((1, 1), lambda i: (0, 0)),
        ],
        out_shape=[
            jax.ShapeDtypeStruct((b, N_EXPERTS, s), jnp.float32),
            jax.ShapeDtypeStruct((b, 2, s), jnp.int32),
            jax.ShapeDtypeStruct((1, 1), jnp.float32),
        ],
        scratch_shapes=[pltpu.VMEM((N_EXPERTS, 1), jnp.float32)],
    )(xf, W)
    gates = jnp.transpose(gates3, (0, 2, 1))
    idx = jnp.transpose(idx3, (0, 2, 1))
    return gates, idx, loss2d[0, 0]


# speedup vs baseline: 12.1240x; 1.2551x over previous
import functools

import jax
import jax.numpy as jnp
from jax.experimental import pallas as pl
from jax.experimental.pallas import tpu as pltpu

D_MODEL = 768
N_EXPERTS = 64
BT = 2048


def _router_body(x_ref, w_ref, gates_ref, idx_ref, loss_ref, counts_ref):
    i = pl.program_id(0)
    nblk = pl.num_programs(0)

    x = x_ref[...]
    w = w_ref[...]
    lt = jax.lax.dot_general(w, x, (((1,), (1,)), ((), ())),
                             preferred_element_type=jnp.float32)

    row = jax.lax.broadcasted_iota(jnp.int32, lt.shape, 0).astype(jnp.float32)
    m1 = jnp.max(lt, axis=0, keepdims=True)
    e1 = jnp.min(jnp.where(lt == m1, row, float(N_EXPERTS)),
                 axis=0, keepdims=True)
    lt2 = jnp.where(row == e1, -jnp.inf, lt)
    m2 = jnp.max(lt2, axis=0, keepdims=True)
    e2 = jnp.min(jnp.where(lt2 == m2, row, float(N_EXPERTS)),
                 axis=0, keepdims=True)

    ed = jnp.exp(m2 - m1)
    s1 = 1.0 / (1.0 + ed)
    s2 = ed * s1

    gates_t = (jnp.where(row == e1, s1, 0.0)
               + jnp.where(row == e2, s2, 0.0))
    gates_ref[...] = gates_t[None]

    idx_t = jnp.concatenate([e1, e2], axis=0).astype(jnp.int32)
    idx_ref[...] = idx_t[None]

    @pl.when(i == 0)
    def _init():
        counts_ref[...] = jnp.zeros_like(counts_ref)

    counts_ref[...] += jnp.sum(gates_t, axis=1, keepdims=True)

    @pl.when(i == nblk - 1)
    def _finish():
        counts = counts_ref[...]
        total = jnp.sum(counts)
        dev = counts / total * N_EXPERTS - 1.0
        loss_ref[...] = jnp.mean(dev * dev, axis=0, keepdims=True)


@functools.partial(jax.jit, static_argnums=())
def kernel(x, W, n_active, capacity):
    b, s, d = x.shape
    t = b * s
    blk_per_batch = s // BT
    xf = x.reshape(t, d)
    grid = (t // BT,)
    gates3, idx3, loss2d = pl.pallas_call(
        _router_body,
        grid=grid,
        in_specs=[
            pl.BlockSpec((BT, D_MODEL), lambda i: (i, 0)),
            pl.BlockSpec((N_EXPERTS, D_MODEL), lambda i: (0, 0)),
        ],
        out_specs=[
            pl.BlockSpec((1, N_EXPERTS, BT),
                         lambda i: (i // blk_per_batch, 0, i % blk_per_batch)),
            pl.BlockSpec((1, 2, BT),
                         lambda i: (i // blk_per_batch, 0, i % blk_per_batch)),
            pl.BlockSpec((1, 1), lambda i: (0, 0)),
        ],
        out_shape=[
            jax.ShapeDtypeStruct((b, N_EXPERTS, s), jnp.float32),
            jax.ShapeDtypeStruct((b, 2, s), jnp.int32),
            jax.ShapeDtypeStruct((1, 1), jnp.float32),
        ],
        scratch_shapes=[pltpu.VMEM((N_EXPERTS, 1), jnp.float32)],
    )(xf, W)
    gates = jnp.transpose(gates3, (0, 2, 1))
    idx = jnp.transpose(idx3, (0, 2, 1))
    return gates, idx, loss2d[0, 0]


# speedup vs baseline: 13.0006x; 1.0723x over previous
import functools

import jax
import jax.numpy as jnp
from jax.experimental import pallas as pl
from jax.experimental.pallas import tpu as pltpu

D_MODEL = 768
N_EXPERTS = 64
BT = 4096


def _router_body(x_ref, w_ref, gates_ref, idx_ref, loss_ref, counts_ref):
    i = pl.program_id(0)
    nblk = pl.num_programs(0)

    x = x_ref[...]
    w = w_ref[...]
    lt = jax.lax.dot_general(w, x, (((1,), (1,)), ((), ())),
                             preferred_element_type=jnp.float32)

    row = jax.lax.broadcasted_iota(jnp.int32, lt.shape, 0).astype(jnp.float32)
    m1 = jnp.max(lt, axis=0, keepdims=True)
    e1 = jnp.min(jnp.where(lt == m1, row, float(N_EXPERTS)),
                 axis=0, keepdims=True)
    lt2 = jnp.where(row == e1, -jnp.inf, lt)
    m2 = jnp.max(lt2, axis=0, keepdims=True)
    e2 = jnp.min(jnp.where(lt2 == m2, row, float(N_EXPERTS)),
                 axis=0, keepdims=True)

    ed = jnp.exp(m2 - m1)
    s1 = 1.0 / (1.0 + ed)
    s2 = ed * s1

    gates_t = (jnp.where(row == e1, s1, 0.0)
               + jnp.where(row == e2, s2, 0.0))
    gates_ref[...] = gates_t[None]

    idx_t = jnp.concatenate([e1, e2], axis=0).astype(jnp.int32)
    idx_ref[...] = idx_t[None]

    @pl.when(i == 0)
    def _init():
        counts_ref[...] = jnp.zeros_like(counts_ref)

    counts_ref[...] += jnp.sum(gates_t, axis=1, keepdims=True)

    @pl.when(i == nblk - 1)
    def _finish():
        counts = counts_ref[...]
        total = jnp.sum(counts)
        dev = counts / total * N_EXPERTS - 1.0
        loss_ref[...] = jnp.mean(dev * dev, axis=0, keepdims=True)


@functools.partial(jax.jit, static_argnums=())
def kernel(x, W, n_active, capacity):
    b, s, d = x.shape
    t = b * s
    blk_per_batch = s // BT
    xf = x.reshape(t, d)
    grid = (t // BT,)
    gates3, idx3, loss2d = pl.pallas_call(
        _router_body,
        grid=grid,
        in_specs=[
            pl.BlockSpec((BT, D_MODEL), lambda i: (i, 0)),
            pl.BlockSpec((N_EXPERTS, D_MODEL), lambda i: (0, 0)),
        ],
        out_specs=[
            pl.BlockSpec((1, N_EXPERTS, BT),
                         lambda i: (i // blk_per_batch, 0, i % blk_per_batch)),
            pl.BlockSpec((1, 2, BT),
                         lambda i: (i // blk_per_batch, 0, i % blk_per_batch)),
            pl.BlockSpec((1, 1), lambda i: (0, 0)),
        ],
        out_shape=[
            jax.ShapeDtypeStruct((b, N_EXPERTS, s), jnp.float32),
            jax.ShapeDtypeStruct((b, 2, s), jnp.int32),
            jax.ShapeDtypeStruct((1, 1), jnp.float32),
        ],
        scratch_shapes=[pltpu.VMEM((N_EXPERTS, 1), jnp.float32)],
    )(xf, W)
    gates = jnp.transpose(gates3, (0, 2, 1))
    idx = jnp.transpose(idx3, (0, 2, 1))
    return gates, idx, loss2d[0, 0]
